# baseline (speedup 1.0000x reference)
import math

import jax
import jax.numpy as jnp
from jax.experimental import pallas as pl
from jax.experimental.pallas import tpu as pltpu

_LOG1E4 = math.log(10000.0)
_HALF_PI = math.pi / 2.0
_H = 64
_SEED = 16
_HI_PER_STEP = 8


def _make_body(d_model):
    def body(o_ref, s1_ref, c1_ref, f64_ref):
        i = pl.program_id(0)

        @pl.when(i == 0)
        def _init():
            col = jax.lax.broadcasted_iota(jnp.int32, (1, d_model), 1)
            parity = col % 2
            k2 = (col - parity).astype(jnp.float32)
            freq = jnp.exp(k2 * (-_LOG1E4 / d_model))
            phase = parity.astype(jnp.float32) * _HALF_PI
            f64_ref[...] = freq * float(_H)
            r = jax.lax.broadcasted_iota(jnp.int32, (_SEED, d_model), 0)
            rf = r.astype(jnp.float32)
            b = (rf + 1.0) * freq + phase
            s1_ref[0:_SEED, :] = jnp.sin(b)
            c1_ref[0:_SEED, :] = jnp.sin(b + _HALF_PI)
            s16 = jnp.sin(freq * float(_SEED))
            c16 = jnp.sin(freq * float(_SEED) + _HALF_PI)
            s0 = s1_ref[0:_SEED, :]
            c0 = c1_ref[0:_SEED, :]
            s1_ref[_SEED : 2 * _SEED, :] = s0 * c16 + c0 * s16
            c1_ref[_SEED : 2 * _SEED, :] = c0 * c16 - s0 * s16
            s32 = 2.0 * s16 * c16
            c32 = c16 * c16 - s16 * s16
            sh = s1_ref[0 : 2 * _SEED, :]
            ch = c1_ref[0 : 2 * _SEED, :]
            s1_ref[2 * _SEED : 4 * _SEED, :] = sh * c32 + ch * s32
            c1_ref[2 * _SEED : 4 * _SEED, :] = ch * c32 - sh * s32

        hr = jax.lax.broadcasted_iota(jnp.int32, (_HI_PER_STEP, d_model), 0)
        hf = (hr + i * _HI_PER_STEP).astype(jnp.float32)
        a = hf * f64_ref[...]
        s2 = jnp.sin(a)[:, None, :]
        c2 = jnp.sin(a + _HALF_PI)[:, None, :]
        s1 = s1_ref[...][None]
        c1 = c1_ref[...][None]
        blk = s2 * c1 + c2 * s1
        o_ref[...] = blk.reshape(_HI_PER_STEP * _H, d_model)

    return body


def kernel(x, pe):
    seq_len = x.shape[-1]
    d_model = pe.shape[-1]
    rows_per_step = _HI_PER_STEP * _H
    grid = (seq_len // rows_per_step,)
    scratch = [
        pltpu.VMEM((_H, d_model), jnp.float32),
        pltpu.VMEM((_H, d_model), jnp.float32),
        pltpu.VMEM((1, d_model), jnp.float32),
    ]
    out2 = pl.pallas_call(
        _make_body(d_model),
        grid=grid,
        out_specs=pl.BlockSpec((rows_per_step, d_model), lambda i: (i, 0)),
        out_shape=jax.ShapeDtypeStruct((seq_len, d_model), pe.dtype),
        scratch_shapes=scratch,
    )()
    return out2[None]


# speedup vs baseline: 1.1165x; 1.1165x over previous
import math

import jax
import jax.numpy as jnp
from jax.experimental import pallas as pl
from jax.experimental.pallas import tpu as pltpu

_LOG1E4 = math.log(10000.0)
_HALF_PI = math.pi / 2.0
_H = 64
_SEED = 8
_HI_PER_STEP = 16


def _make_body(d_model):
    def body(o_ref, s1_ref, c1_ref, f64_ref):
        i = pl.program_id(0)

        @pl.when(i == 0)
        def _init():
            col = jax.lax.broadcasted_iota(jnp.int32, (1, d_model), 1)
            parity = col % 2
            k2 = (col - parity).astype(jnp.float32)
            freq = jnp.exp(k2 * (-_LOG1E4 / d_model))
            phase = parity.astype(jnp.float32) * _HALF_PI
            f64_ref[...] = freq * float(_H)
            r = jax.lax.broadcasted_iota(jnp.int32, (_SEED, d_model), 0)
            rf = r.astype(jnp.float32)
            b = (rf + 1.0) * freq + phase
            s1_ref[0:_SEED, :] = jnp.sin(b)
            c1_ref[0:_SEED, :] = jnp.sin(b + _HALF_PI)
            sk = jnp.sin(freq * float(_SEED))
            ck = jnp.sin(freq * float(_SEED) + _HALF_PI)
            n = _SEED
            for _ in range(3):
                s0 = s1_ref[0:n, :]
                c0 = c1_ref[0:n, :]
                s1_ref[n : 2 * n, :] = s0 * ck + c0 * sk
                c1_ref[n : 2 * n, :] = c0 * ck - s0 * sk
                sk, ck = 2.0 * sk * ck, ck * ck - sk * sk
                n *= 2

        hr = jax.lax.broadcasted_iota(jnp.int32, (_HI_PER_STEP, d_model), 0)
        hf = (hr + i * _HI_PER_STEP).astype(jnp.float32)
        a = hf * f64_ref[...]
        s2 = jnp.sin(a)[:, None, :]
        c2 = jnp.sin(a + _HALF_PI)[:, None, :]
        s1 = s1_ref[...][None]
        c1 = c1_ref[...][None]
        blk = s2 * c1 + c2 * s1
        o_ref[...] = blk.reshape(_HI_PER_STEP * _H, d_model)

    return body


def kernel(x, pe):
    seq_len = x.shape[-1]
    d_model = pe.shape[-1]
    rows_per_step = _HI_PER_STEP * _H
    grid = (seq_len // rows_per_step,)
    scratch = [
        pltpu.VMEM((_H, d_model), jnp.float32),
        pltpu.VMEM((_H, d_model), jnp.float32),
        pltpu.VMEM((1, d_model), jnp.float32),
    ]
    out2 = pl.pallas_call(
        _make_body(d_model),
        grid=grid,
        out_specs=pl.BlockSpec((rows_per_step, d_model), lambda i: (i, 0)),
        out_shape=jax.ShapeDtypeStruct((seq_len, d_model), pe.dtype),
        scratch_shapes=scratch,
    )()
    return out2[None]
